# Initial kernel scaffold; baseline (speedup 1.0000x reference)
import jax
import jax.numpy as jnp
from jax.experimental import pallas as pl


def kernel(inputs, p):
    raise NotImplementedError("write your pallas kernel here")


# speedup vs baseline: 308.7611x; 308.7611x over previous
import math

import numpy as np
import jax
import jax.numpy as jnp
from jax import lax
from jax.experimental import pallas as pl
from jax.experimental.pallas import tpu as pltpu
from jax.experimental.pallas import tpu_sc as plsc

N_BINS = 32
BOUND = 30.0
R = 1.2
BETA = 1e-08
DIM = 64
BATCH = 16384
LUT_SIZE = 1024
L = 16
NW = 32
CHUNK = BATCH * DIM // NW


def _build_tables():
    m = N_BINS / 2
    x1L = BOUND * (R - 1.0) / (math.pow(R, m) - 1.0)
    index = np.arange(0, N_BINS + 1, dtype=np.float64) - m
    xr = np.where(index >= 0, (1.0 - np.power(R, index)) / (1.0 - R),
                  (1.0 - np.power(R, np.abs(index))) / (1.0 - R))
    xr = np.where(index >= 0, x1L * xr, -x1L * xr)
    xr = (xr + BOUND) / 2.0 / BOUND
    mesh = np.concatenate([[0.0], xr[1:-1], [1.0]]).astype(np.float32)
    elmt = (mesh[1:] - mesh[:-1]).astype(np.float32)
    u = np.arange(LUT_SIZE, dtype=np.float64) / LUT_SIZE
    lut = np.searchsorted(mesh.astype(np.float64), u, side='right') - 1
    lut = np.clip(lut, 0, N_BINS - 1).astype(np.int32)
    return mesh, elmt, lut


_MESH, _ELMT, _LUT = _build_tables()
_MESH_HI = _MESH[1:].copy()


def _sc_body(x_hbm, p_hbm, lut_hbm, mhi_hbm, out_hbm,
             x_v, p_v, a_v, b_v, c_v, lut_v, mhi_v):
    nc = 2
    wid = lax.axis_index("s") * nc + lax.axis_index("c")
    base = wid * CHUNK

    pltpu.sync_copy(lut_hbm, lut_v)
    pltpu.sync_copy(mhi_hbm, mhi_v)
    pltpu.sync_copy(p_hbm, p_v)
    pltpu.sync_copy(x_hbm.at[pl.ds(base, CHUNK)], x_v)

    f32 = jnp.float32
    for g in range(DIM // L):
        co = g * L
        denom = jnp.zeros((L,), f32)
        for j in range(N_BINS - 1):
            e = jnp.exp(p_v[pl.ds(j * DIM + co, L)])
            denom = denom + e * f32(0.5 * (float(_ELMT[j]) + float(_ELMT[j + 1])))
        scale = f32(1.0 - (float(_ELMT[0]) + float(_ELMT[-1])) * BETA / 2.0) / denom
        frun = jnp.zeros((L,), f32)
        v1 = jnp.full((L,), f32(BETA))
        for k in range(N_BINS):
            if k == N_BINS - 1:
                v2 = jnp.full((L,), f32(BETA))
            else:
                v2 = jnp.exp(p_v[pl.ds(k * DIM + co, L)]) * scale
            hk = float(_ELMT[k])
            mk = float(_MESH[k])
            q = (v2 - v1) * f32(0.5 / hk)
            c_v[pl.ds(k * DIM + co, L)] = q
            b_v[pl.ds(k * DIM + co, L)] = v1 - f32(2.0 * mk) * q
            a_v[pl.ds(k * DIM + co, L)] = frun + f32(mk * mk) * q - f32(mk) * v1
            frun = frun + (v1 + v2) * f32(0.5 * hk)
            v1 = v2

    iota = lax.iota(jnp.int32, L)

    def row_body(i, carry):
        for s in range(DIM // L):
            off = i * DIM + s * L
            xr = x_v[pl.ds(off, L)]
            xn = (xr + f32(BOUND)) / f32(2.0) / f32(BOUND)
            t = jnp.minimum(jnp.maximum(xn * f32(LUT_SIZE), f32(0.0)),
                            f32(LUT_SIZE - 1))
            u = t.astype(jnp.int32)
            k0 = plsc.load_gather(lut_v, [u])
            mhi = plsc.load_gather(mhi_v, [k0])
            k = jnp.minimum(
                k0 + jnp.where(xn >= mhi, jnp.int32(1), jnp.int32(0)),
                jnp.int32(N_BINS - 1))
            flat = k * jnp.int32(DIM) + (iota + jnp.int32(s * L))
            av = plsc.load_gather(a_v, [flat])
            bv = plsc.load_gather(b_v, [flat])
            cv = plsc.load_gather(c_v, [flat])
            y = av + xn * (bv + xn * cv)
            cover = (xn >= f32(0.0)) & (xn < f32(1.0))
            y = jnp.where(cover, y, xn)
            y = y * f32(2.0) * f32(BOUND) - f32(BOUND)
            y = jnp.where(y > f32(BOUND), f32(BETA) * (y - f32(BOUND)) + f32(BOUND), y)
            y = jnp.where(y < -f32(BOUND), f32(BETA) * (y + f32(BOUND)) - f32(BOUND), y)
            x_v[pl.ds(off, L)] = y
        return carry

    lax.fori_loop(0, CHUNK // DIM, row_body, 0)
    pltpu.sync_copy(x_v, out_hbm.at[pl.ds(base, CHUNK)])


@jax.jit
def kernel(inputs, p):
    mesh = plsc.VectorSubcoreMesh(core_axis_name="c", subcore_axis_name="s")
    run = pl.kernel(
        _sc_body,
        out_type=jax.ShapeDtypeStruct((BATCH * DIM,), jnp.float32),
        mesh=mesh,
        scratch_types=[
            pltpu.VMEM((CHUNK,), jnp.float32),
            pltpu.VMEM(((N_BINS - 1) * DIM,), jnp.float32),
            pltpu.VMEM((N_BINS * DIM,), jnp.float32),
            pltpu.VMEM((N_BINS * DIM,), jnp.float32),
            pltpu.VMEM((N_BINS * DIM,), jnp.float32),
            pltpu.VMEM((LUT_SIZE,), jnp.int32),
            pltpu.VMEM((N_BINS,), jnp.float32),
        ],
        compiler_params=pltpu.CompilerParams(needs_layout_passes=False),
    )
    out = run(inputs.reshape(-1), p.reshape(-1),
              jnp.asarray(_LUT), jnp.asarray(_MESH_HI))
    return out.reshape(BATCH, DIM)


# speedup vs baseline: 581.4451x; 1.8832x over previous
import math

import numpy as np
import jax
import jax.numpy as jnp
from jax import lax
from jax.experimental import pallas as pl
from jax.experimental.pallas import tpu as pltpu
from jax.experimental.pallas import tpu_sc as plsc

N_BINS = 32
BOUND = 30.0
R = 1.2
BETA = 1e-08
DIM = 64
BATCH = 16384
LUT_SIZE = 1024
L = 16
NW = 32
CHUNK = BATCH * DIM // NW


def _build_tables():
    m = N_BINS / 2
    x1L = BOUND * (R - 1.0) / (math.pow(R, m) - 1.0)
    index = np.arange(0, N_BINS + 1, dtype=np.float64) - m
    xr = np.where(index >= 0, (1.0 - np.power(R, index)) / (1.0 - R),
                  (1.0 - np.power(R, np.abs(index))) / (1.0 - R))
    xr = np.where(index >= 0, x1L * xr, -x1L * xr)
    xr = (xr + BOUND) / 2.0 / BOUND
    mesh = np.concatenate([[0.0], xr[1:-1], [1.0]]).astype(np.float32)
    elmt = (mesh[1:] - mesh[:-1]).astype(np.float32)
    u = np.arange(LUT_SIZE, dtype=np.float64) / LUT_SIZE
    lut = np.searchsorted(mesh.astype(np.float64), u, side='right') - 1
    lut = np.clip(lut, 0, N_BINS - 1).astype(np.int32)
    return mesh, elmt, lut


_MESH, _ELMT, _LUT = _build_tables()
_MESH_HI = _MESH[1:].copy()
_LUT_MHI = _MESH_HI[_LUT].copy()


def _sc_body(x_hbm, p_hbm, lut_hbm, mhi_hbm, out_hbm,
             x_v, p_v, a_v, b_v, c_v, lut_v, mhi_v):
    nc = 2
    wid = lax.axis_index("s") * nc + lax.axis_index("c")
    base = wid * CHUNK

    pltpu.sync_copy(lut_hbm, lut_v)
    pltpu.sync_copy(mhi_hbm, mhi_v)
    pltpu.sync_copy(p_hbm, p_v)
    pltpu.sync_copy(x_hbm.at[pl.ds(base, CHUNK)], x_v)

    f32 = jnp.float32
    for g in range(DIM // L):
        co = g * L
        denom = jnp.zeros((L,), f32)
        for j in range(N_BINS - 1):
            e = jnp.exp(p_v[pl.ds(j * DIM + co, L)])
            denom = denom + e * f32(0.5 * (float(_ELMT[j]) + float(_ELMT[j + 1])))
        scale = f32(1.0 - (float(_ELMT[0]) + float(_ELMT[-1])) * BETA / 2.0) / denom
        frun = jnp.zeros((L,), f32)
        v1 = jnp.full((L,), f32(BETA))
        for k in range(N_BINS):
            if k == N_BINS - 1:
                v2 = jnp.full((L,), f32(BETA))
            else:
                v2 = jnp.exp(p_v[pl.ds(k * DIM + co, L)]) * scale
            hk = float(_ELMT[k])
            mk = float(_MESH[k])
            q = (v2 - v1) * f32(0.5 / hk)
            c_v[pl.ds(k * DIM + co, L)] = q
            b_v[pl.ds(k * DIM + co, L)] = v1 - f32(2.0 * mk) * q
            a_v[pl.ds(k * DIM + co, L)] = frun + f32(mk * mk) * q - f32(mk) * v1
            frun = frun + (v1 + v2) * f32(0.5 * hk)
            v1 = v2

    iota = lax.iota(jnp.int32, L)

    @plsc.parallel_loop(0, CHUNK // DIM, 1, unroll=2)
    def _row(i):
        for s in range(DIM // L):
            off = i * DIM + s * L
            xr = x_v[pl.ds(off, L)]
            xn = (xr + f32(BOUND)) * f32(1.0 / (2.0 * BOUND))
            t = jnp.minimum(jnp.maximum(xn * f32(LUT_SIZE), f32(0.0)),
                            f32(LUT_SIZE - 1))
            u = t.astype(jnp.int32)
            k0 = plsc.load_gather(lut_v, [u])
            mhi = plsc.load_gather(mhi_v, [u])
            k = jnp.minimum(
                k0 + jnp.where(xn >= mhi, jnp.int32(1), jnp.int32(0)),
                jnp.int32(N_BINS - 1))
            flat = k * jnp.int32(DIM) + (iota + jnp.int32(s * L))
            av = plsc.load_gather(a_v, [flat])
            bv = plsc.load_gather(b_v, [flat])
            cv = plsc.load_gather(c_v, [flat])
            y = av + xn * (bv + xn * cv)
            cover = (xn >= f32(0.0)) & (xn < f32(1.0))
            y = jnp.where(cover, y, xn)
            y = y * f32(2.0 * BOUND) - f32(BOUND)
            yc = jnp.minimum(jnp.maximum(y, f32(-BOUND)), f32(BOUND))
            y = yc + f32(BETA) * (y - yc)
            x_v[pl.ds(off, L)] = y

    pltpu.sync_copy(x_v, out_hbm.at[pl.ds(base, CHUNK)])


@jax.jit
def kernel(inputs, p):
    mesh = plsc.VectorSubcoreMesh(core_axis_name="c", subcore_axis_name="s")
    run = pl.kernel(
        _sc_body,
        out_type=jax.ShapeDtypeStruct((BATCH * DIM,), jnp.float32),
        mesh=mesh,
        scratch_types=[
            pltpu.VMEM((CHUNK,), jnp.float32),
            pltpu.VMEM(((N_BINS - 1) * DIM,), jnp.float32),
            pltpu.VMEM((N_BINS * DIM,), jnp.float32),
            pltpu.VMEM((N_BINS * DIM,), jnp.float32),
            pltpu.VMEM((N_BINS * DIM,), jnp.float32),
            pltpu.VMEM((LUT_SIZE,), jnp.int32),
            pltpu.VMEM((LUT_SIZE,), jnp.float32),
        ],
        compiler_params=pltpu.CompilerParams(needs_layout_passes=False),
    )
    out = run(inputs.reshape(-1), p.reshape(-1),
              jnp.asarray(_LUT), jnp.asarray(_LUT_MHI))
    return out.reshape(BATCH, DIM)


# speedup vs baseline: 583.8319x; 1.0041x over previous
import math

import numpy as np
import jax
import jax.numpy as jnp
from jax import lax
from jax.experimental import pallas as pl
from jax.experimental.pallas import tpu as pltpu
from jax.experimental.pallas import tpu_sc as plsc

N_BINS = 32
BOUND = 30.0
R = 1.2
BETA = 1e-08
DIM = 64
BATCH = 16384
LUT_SIZE = 1024
L = 16
NW = 32
CHUNK = BATCH * DIM // NW


def _build_tables():
    m = N_BINS / 2
    x1L = BOUND * (R - 1.0) / (math.pow(R, m) - 1.0)
    index = np.arange(0, N_BINS + 1, dtype=np.float64) - m
    xr = np.where(index >= 0, (1.0 - np.power(R, index)) / (1.0 - R),
                  (1.0 - np.power(R, np.abs(index))) / (1.0 - R))
    xr = np.where(index >= 0, x1L * xr, -x1L * xr)
    xr = (xr + BOUND) / 2.0 / BOUND
    mesh = np.concatenate([[0.0], xr[1:-1], [1.0]]).astype(np.float32)
    elmt = (mesh[1:] - mesh[:-1]).astype(np.float32)
    u = np.arange(LUT_SIZE, dtype=np.float64) / LUT_SIZE
    lut = np.searchsorted(mesh.astype(np.float64), u, side='right') - 1
    lut = np.clip(lut, 0, N_BINS - 1).astype(np.int32)
    return mesh, elmt, lut


_MESH, _ELMT, _LUT = _build_tables()
_MESH_HI = _MESH[1:].copy()
_LUT_MHI = _MESH_HI[_LUT].copy()


def _sc_body(x_hbm, p_hbm, lut_hbm, mhi_hbm, out_hbm,
             x_v, y_v, p_v, a_v, b_v, c_v, lut_v, mhi_v):
    nc = 2
    wid = lax.axis_index("s") * nc + lax.axis_index("c")
    base = wid * CHUNK

    pltpu.sync_copy(lut_hbm, lut_v)
    pltpu.sync_copy(mhi_hbm, mhi_v)
    pltpu.sync_copy(p_hbm, p_v)
    pltpu.sync_copy(x_hbm.at[pl.ds(base, CHUNK)], x_v)

    f32 = jnp.float32
    for g in range(DIM // L):
        co = g * L
        denom = jnp.zeros((L,), f32)
        for j in range(N_BINS - 1):
            e = jnp.exp(p_v[pl.ds(j * DIM + co, L)])
            denom = denom + e * f32(0.5 * (float(_ELMT[j]) + float(_ELMT[j + 1])))
        scale = f32(1.0 - (float(_ELMT[0]) + float(_ELMT[-1])) * BETA / 2.0) / denom
        frun = jnp.zeros((L,), f32)
        v1 = jnp.full((L,), f32(BETA))
        for k in range(N_BINS):
            if k == N_BINS - 1:
                v2 = jnp.full((L,), f32(BETA))
            else:
                v2 = jnp.exp(p_v[pl.ds(k * DIM + co, L)]) * scale
            hk = float(_ELMT[k])
            mk = float(_MESH[k])
            q = (v2 - v1) * f32(0.5 / hk)
            c_v[pl.ds(k * DIM + co, L)] = q
            b_v[pl.ds(k * DIM + co, L)] = v1 - f32(2.0 * mk) * q
            a_v[pl.ds(k * DIM + co, L)] = frun + f32(mk * mk) * q - f32(mk) * v1
            frun = frun + (v1 + v2) * f32(0.5 * hk)
            v1 = v2

    iota = lax.iota(jnp.int32, L)

    @plsc.parallel_loop(0, CHUNK // DIM, 1, unroll=4)
    def _row(i):
        for s in range(DIM // L):
            off = i * DIM + s * L
            xr = x_v[pl.ds(off, L)]
            xn = (xr + f32(BOUND)) * f32(1.0 / (2.0 * BOUND))
            t = jnp.minimum(jnp.maximum(xn * f32(LUT_SIZE), f32(0.0)),
                            f32(LUT_SIZE - 1))
            u = t.astype(jnp.int32)
            k0 = plsc.load_gather(lut_v, [u])
            mhi = plsc.load_gather(mhi_v, [u])
            k = jnp.minimum(
                k0 + jnp.where(xn >= mhi, jnp.int32(1), jnp.int32(0)),
                jnp.int32(N_BINS - 1))
            flat = k * jnp.int32(DIM) + (iota + jnp.int32(s * L))
            av = plsc.load_gather(a_v, [flat])
            bv = plsc.load_gather(b_v, [flat])
            cv = plsc.load_gather(c_v, [flat])
            y = av + xn * (bv + xn * cv)
            cover = (xn >= f32(0.0)) & (xn < f32(1.0))
            y = jnp.where(cover, y, xn)
            y = y * f32(2.0 * BOUND) - f32(BOUND)
            yc = jnp.minimum(jnp.maximum(y, f32(-BOUND)), f32(BOUND))
            y = yc + f32(BETA) * (y - yc)
            y_v[pl.ds(off, L)] = y

    pltpu.sync_copy(y_v, out_hbm.at[pl.ds(base, CHUNK)])


@jax.jit
def kernel(inputs, p):
    mesh = plsc.VectorSubcoreMesh(core_axis_name="c", subcore_axis_name="s")
    run = pl.kernel(
        _sc_body,
        out_type=jax.ShapeDtypeStruct((BATCH * DIM,), jnp.float32),
        mesh=mesh,
        scratch_types=[
            pltpu.VMEM((CHUNK,), jnp.float32),
            pltpu.VMEM((CHUNK,), jnp.float32),
            pltpu.VMEM(((N_BINS - 1) * DIM,), jnp.float32),
            pltpu.VMEM((N_BINS * DIM,), jnp.float32),
            pltpu.VMEM((N_BINS * DIM,), jnp.float32),
            pltpu.VMEM((N_BINS * DIM,), jnp.float32),
            pltpu.VMEM((LUT_SIZE,), jnp.int32),
            pltpu.VMEM((LUT_SIZE,), jnp.float32),
        ],
        compiler_params=pltpu.CompilerParams(needs_layout_passes=False),
    )
    out = run(inputs.reshape(-1), p.reshape(-1),
              jnp.asarray(_LUT), jnp.asarray(_LUT_MHI))
    return out.reshape(BATCH, DIM)


# speedup vs baseline: 713.9133x; 1.2228x over previous
import math

import numpy as np
import jax
import jax.numpy as jnp
from jax import lax
from jax.experimental import pallas as pl
from jax.experimental.pallas import tpu as pltpu
from jax.experimental.pallas import tpu_sc as plsc

N_BINS = 32
BOUND = 30.0
R = 1.2
BETA = 1e-08
DIM = 64
BATCH = 16384
LUT_SIZE = 1024
L = 16
NW = 32
ROWS = BATCH // NW


def _build_tables():
    m = N_BINS / 2
    x1L = BOUND * (R - 1.0) / (math.pow(R, m) - 1.0)
    index = np.arange(0, N_BINS + 1, dtype=np.float64) - m
    xr = np.where(index >= 0, (1.0 - np.power(R, index)) / (1.0 - R),
                  (1.0 - np.power(R, np.abs(index))) / (1.0 - R))
    xr = np.where(index >= 0, x1L * xr, -x1L * xr)
    xr = (xr + BOUND) / 2.0 / BOUND
    mesh = np.concatenate([[0.0], xr[1:-1], [1.0]]).astype(np.float32)
    elmt = (mesh[1:] - mesh[:-1]).astype(np.float32)
    u = np.arange(LUT_SIZE, dtype=np.float64) / LUT_SIZE
    lut = np.searchsorted(mesh.astype(np.float64), u, side='right') - 1
    lut = np.clip(lut, 0, N_BINS - 1).astype(np.int32)
    return mesh, elmt, lut


_MESH, _ELMT, _LUT = _build_tables()
_MESH_HI = _MESH[1:].copy()
_LUT_MHI = _MESH_HI[_LUT].copy()


def _sc_body(x_hbm, p_hbm, lut_hbm, mhi_hbm, out_hbm,
             x_v, p_v, a_v, b_v, c_v, lut_v, mhi_v):
    nc = 2
    wid = lax.axis_index("s") * nc + lax.axis_index("c")
    base = wid * ROWS

    pltpu.sync_copy(lut_hbm, lut_v)
    pltpu.sync_copy(mhi_hbm, mhi_v)
    pltpu.sync_copy(p_hbm, p_v)
    pltpu.sync_copy(x_hbm.at[pl.ds(base, ROWS)], x_v)

    f32 = jnp.float32
    for g in range(DIM // L):
        co = g * L
        denom = jnp.zeros((L,), f32)
        for j in range(N_BINS - 1):
            e = jnp.exp(p_v[j, pl.ds(co, L)])
            denom = denom + e * f32(0.5 * (float(_ELMT[j]) + float(_ELMT[j + 1])))
        scale = f32(1.0 - (float(_ELMT[0]) + float(_ELMT[-1])) * BETA / 2.0) / denom
        frun = jnp.zeros((L,), f32)
        v1 = jnp.full((L,), f32(BETA))
        for k in range(N_BINS):
            if k == N_BINS - 1:
                v2 = jnp.full((L,), f32(BETA))
            else:
                v2 = jnp.exp(p_v[k, pl.ds(co, L)]) * scale
            hk = float(_ELMT[k])
            mk = float(_MESH[k])
            q = (v2 - v1) * f32(0.5 / hk)
            c_v[pl.ds(k * DIM + co, L)] = q
            b_v[pl.ds(k * DIM + co, L)] = v1 - f32(2.0 * mk) * q
            a_v[pl.ds(k * DIM + co, L)] = frun + f32(mk * mk) * q - f32(mk) * v1
            frun = frun + (v1 + v2) * f32(0.5 * hk)
            v1 = v2

    iota = lax.iota(jnp.int32, L)

    @plsc.parallel_loop(0, ROWS, 1, unroll=4)
    def _row(i):
        for s in range(DIM // L):
            cs = pl.ds(s * L, L)
            xr = x_v[i, cs]
            xn = (xr + f32(BOUND)) * f32(1.0 / (2.0 * BOUND))
            t = jnp.minimum(jnp.maximum(xn * f32(LUT_SIZE), f32(0.0)),
                            f32(LUT_SIZE - 1))
            u = t.astype(jnp.int32)
            k0 = plsc.load_gather(lut_v, [u])
            mhi = plsc.load_gather(mhi_v, [u])
            k = jnp.minimum(
                k0 + jnp.where(xn >= mhi, jnp.int32(1), jnp.int32(0)),
                jnp.int32(N_BINS - 1))
            flat = k * jnp.int32(DIM) + (iota + jnp.int32(s * L))
            av = plsc.load_gather(a_v, [flat])
            bv = plsc.load_gather(b_v, [flat])
            cv = plsc.load_gather(c_v, [flat])
            y = av + xn * (bv + xn * cv)
            cover = (xn >= f32(0.0)) & (xn < f32(1.0))
            y = jnp.where(cover, y, xn)
            y = y * f32(2.0 * BOUND) - f32(BOUND)
            yc = jnp.minimum(jnp.maximum(y, f32(-BOUND)), f32(BOUND))
            y = yc + f32(BETA) * (y - yc)
            x_v[i, cs] = y

    pltpu.sync_copy(x_v, out_hbm.at[pl.ds(base, ROWS)])


@jax.jit
def kernel(inputs, p):
    mesh = plsc.VectorSubcoreMesh(core_axis_name="c", subcore_axis_name="s")
    run = pl.kernel(
        _sc_body,
        out_type=jax.ShapeDtypeStruct((BATCH, DIM), jnp.float32),
        mesh=mesh,
        scratch_types=[
            pltpu.VMEM((ROWS, DIM), jnp.float32),
            pltpu.VMEM((N_BINS - 1, DIM), jnp.float32),
            pltpu.VMEM((N_BINS * DIM,), jnp.float32),
            pltpu.VMEM((N_BINS * DIM,), jnp.float32),
            pltpu.VMEM((N_BINS * DIM,), jnp.float32),
            pltpu.VMEM((LUT_SIZE,), jnp.int32),
            pltpu.VMEM((LUT_SIZE,), jnp.float32),
        ],
        compiler_params=pltpu.CompilerParams(needs_layout_passes=False),
    )
    return run(inputs, p, jnp.asarray(_LUT), jnp.asarray(_LUT_MHI))


# speedup vs baseline: 813.3188x; 1.1392x over previous
import math

import numpy as np
import jax
import jax.numpy as jnp
from jax import lax
from jax.experimental import pallas as pl
from jax.experimental.pallas import tpu as pltpu
from jax.experimental.pallas import tpu_sc as plsc

N_BINS = 32
BOUND = 30.0
R = 1.2
BETA = 1e-08
DIM = 64
BATCH = 16384
LUT2 = 4096
LUT_N = 4104
EXT = N_BINS + 2
L = 16
NW = 32
ROWS = BATCH // NW


def _build_tables():
    m = N_BINS / 2
    x1L = BOUND * (R - 1.0) / (math.pow(R, m) - 1.0)
    index = np.arange(0, N_BINS + 1, dtype=np.float64) - m
    xr = np.where(index >= 0, (1.0 - np.power(R, index)) / (1.0 - R),
                  (1.0 - np.power(R, np.abs(index))) / (1.0 - R))
    xr = np.where(index >= 0, x1L * xr, -x1L * xr)
    xr = (xr + BOUND) / 2.0 / BOUND
    mesh = np.concatenate([[0.0], xr[1:-1], [1.0]]).astype(np.float32)
    elmt = (mesh[1:] - mesh[:-1]).astype(np.float32)
    ext = np.empty(LUT_N, np.int64)
    ext[0] = 0
    ext[LUT2 + 1:] = EXT - 1
    mid = (np.arange(1, LUT2 + 1) - 0.5) / LUT2
    b = np.searchsorted(mesh.astype(np.float64), mid, side='right') - 1
    ext[1:LUT2 + 1] = np.clip(b, -1, N_BINS) + 1
    lut = (ext * DIM).astype(np.int32)
    return mesh, elmt, lut


_MESH, _ELMT, _LUT = _build_tables()


def _sc_body(x_hbm, p_hbm, lut_hbm, out_hbm,
             x_v, p_v, a_v, b_v, c_v, lut_v, sem_p, sem_l, sem_x):
    nc = 2
    wid = lax.axis_index("s") * nc + lax.axis_index("c")
    base = wid * ROWS

    h_p = pltpu.async_copy(p_hbm, p_v, sem_p)
    h_l = pltpu.async_copy(lut_hbm, lut_v, sem_l)
    h_x = pltpu.async_copy(x_hbm.at[pl.ds(base, ROWS)], x_v, sem_x)
    h_p.wait()

    f32 = jnp.float32
    for g in range(DIM // L):
        co = g * L
        ident_a = jnp.full((L,), f32(-BOUND))
        ident_b = jnp.full((L,), f32(2.0 * BOUND))
        ident_c = jnp.zeros((L,), f32)
        a_v[pl.ds(co, L)] = ident_a
        b_v[pl.ds(co, L)] = ident_b
        c_v[pl.ds(co, L)] = ident_c
        a_v[pl.ds((EXT - 1) * DIM + co, L)] = ident_a
        b_v[pl.ds((EXT - 1) * DIM + co, L)] = ident_b
        c_v[pl.ds((EXT - 1) * DIM + co, L)] = ident_c
        denom = jnp.zeros((L,), f32)
        for j in range(N_BINS - 1):
            e = jnp.exp(p_v[j, pl.ds(co, L)])
            denom = denom + e * f32(0.5 * (float(_ELMT[j]) + float(_ELMT[j + 1])))
        scale = f32(1.0 - (float(_ELMT[0]) + float(_ELMT[-1])) * BETA / 2.0) / denom
        frun = jnp.zeros((L,), f32)
        v1 = jnp.full((L,), f32(BETA))
        for k in range(N_BINS):
            if k == N_BINS - 1:
                v2 = jnp.full((L,), f32(BETA))
            else:
                v2 = jnp.exp(p_v[k, pl.ds(co, L)]) * scale
            hk = float(_ELMT[k])
            mk = float(_MESH[k])
            q = (v2 - v1) * f32(0.5 / hk)
            row = (k + 1) * DIM + co
            c_v[pl.ds(row, L)] = q * f32(2.0 * BOUND)
            b_v[pl.ds(row, L)] = (v1 - f32(2.0 * mk) * q) * f32(2.0 * BOUND)
            a_v[pl.ds(row, L)] = (frun + f32(mk * mk) * q - f32(mk) * v1) \
                * f32(2.0 * BOUND) - f32(BOUND)
            frun = frun + (v1 + v2) * f32(0.5 * hk)
            v1 = v2

    iota = lax.iota(jnp.int32, L)
    h_l.wait()
    h_x.wait()

    @plsc.parallel_loop(0, ROWS, 1, unroll=4)
    def _row(i):
        for s in range(DIM // L):
            cs = pl.ds(s * L, L)
            xr = x_v[i, cs]
            xn = (xr + f32(BOUND)) * f32(1.0 / (2.0 * BOUND))
            t = jnp.minimum(jnp.maximum(xn * f32(LUT2), f32(-1.0)), f32(LUT2))
            u = (t + f32(1.0)).astype(jnp.int32)
            fk = plsc.load_gather(lut_v, [u])
            flat = fk + (iota + jnp.int32(s * L))
            av = plsc.load_gather(a_v, [flat])
            bv = plsc.load_gather(b_v, [flat])
            cv = plsc.load_gather(c_v, [flat])
            y = av + xn * (bv + xn * cv)
            yc = jnp.minimum(jnp.maximum(y, f32(-BOUND)), f32(BOUND))
            y = yc + f32(BETA) * (y - yc)
            x_v[i, cs] = y

    pltpu.sync_copy(x_v, out_hbm.at[pl.ds(base, ROWS)])


@jax.jit
def kernel(inputs, p):
    mesh = plsc.VectorSubcoreMesh(core_axis_name="c", subcore_axis_name="s")
    run = pl.kernel(
        _sc_body,
        out_type=jax.ShapeDtypeStruct((BATCH, DIM), jnp.float32),
        mesh=mesh,
        scratch_types=[
            pltpu.VMEM((ROWS, DIM), jnp.float32),
            pltpu.VMEM((N_BINS - 1, DIM), jnp.float32),
            pltpu.VMEM((EXT * DIM,), jnp.float32),
            pltpu.VMEM((EXT * DIM,), jnp.float32),
            pltpu.VMEM((EXT * DIM,), jnp.float32),
            pltpu.VMEM((LUT_N,), jnp.int32),
            pltpu.SemaphoreType.DMA,
            pltpu.SemaphoreType.DMA,
            pltpu.SemaphoreType.DMA,
        ],
        compiler_params=pltpu.CompilerParams(needs_layout_passes=False),
    )
    return run(inputs, p, jnp.asarray(_LUT))
